# Initial kernel scaffold; baseline (speedup 1.0000x reference)
import jax
import jax.numpy as jnp
from jax.experimental import pallas as pl


def kernel(x, expert_indices, expert_weights, gate_proj, up_proj, down_proj):
    raise NotImplementedError("write your pallas kernel here")


# speedup vs baseline: 3.4646x; 3.4646x over previous
import functools

import jax
import jax.numpy as jnp
from jax.experimental import pallas as pl
from jax.experimental.pallas import tpu as pltpu

HID = 1024
INT = 4096
NE = 8
ITILE = 512
NIT = INT // ITILE


def _moe_dense_body(ei_ref, ew_ref, x_ref, g_ref, u_ref, d_ref, o_ref):
    e = pl.program_id(0)
    it = pl.program_id(1)
    x = x_ref[...]
    g = g_ref[0]
    u = u_ref[0]
    d = d_ref[0]
    gate = jax.lax.dot_general(x, g, (((1,), (1,)), ((), ())),
                               preferred_element_type=jnp.float32)
    up = jax.lax.dot_general(x, u, (((1,), (1,)), ((), ())),
                             preferred_element_type=jnp.float32)
    inter = gate * jax.nn.sigmoid(gate) * up
    part = jax.lax.dot_general(inter, d, (((1,), (1,)), ((), ())),
                               preferred_element_type=jnp.float32)
    ei = ei_ref[...]
    ew = ew_ref[...]
    w = jnp.sum(jnp.where(ei == e, ew, 0.0), axis=0)
    part = part * w[:, None]

    @pl.when(jnp.logical_and(e == 0, it == 0))
    def _init():
        o_ref[...] = jnp.zeros_like(o_ref)

    o_ref[...] += part


def kernel(x, expert_indices, expert_weights, gate_proj, up_proj, down_proj):
    batch, seq, hid = x.shape
    x_flat = x.reshape(seq, hid)
    ei = expert_indices.reshape(seq, 2).T.astype(jnp.int32)
    ew = expert_weights.reshape(seq, 2).T
    grid = (NE, NIT)
    out = pl.pallas_call(
        _moe_dense_body,
        grid=grid,
        in_specs=[
            pl.BlockSpec((2, seq), lambda e, it: (0, 0)),
            pl.BlockSpec((2, seq), lambda e, it: (0, 0)),
            pl.BlockSpec((seq, hid), lambda e, it: (0, 0)),
            pl.BlockSpec((1, ITILE, HID), lambda e, it: (e, it, 0)),
            pl.BlockSpec((1, ITILE, HID), lambda e, it: (e, it, 0)),
            pl.BlockSpec((1, HID, ITILE), lambda e, it: (e, 0, it)),
        ],
        out_specs=pl.BlockSpec((seq, hid), lambda e, it: (0, 0)),
        out_shape=jax.ShapeDtypeStruct((seq, hid), jnp.float32),
        compiler_params=pltpu.CompilerParams(
            dimension_semantics=("arbitrary", "arbitrary"),
        ),
    )(ei, ew, x_flat, gate_proj, up_proj, down_proj)
    return out.reshape(batch, seq, hid)
